# baseline (speedup 1.0000x reference)
import jax
import jax.numpy as jnp
from jax.experimental import pallas as pl
from jax.experimental.pallas import tpu as pltpu

VOCAB = 1000
HIDDEN = 128
BATCH = 1024
SEQ = 20
TOKENS = BATCH * SEQ

BB = 64
TB = BB * SEQ


def _body(ids_ref, e_ref, pos_ref, wt_ref, out_ref):
    ids = ids_ref[...]
    iota = jax.lax.broadcasted_iota(jnp.int32, (TB, VOCAB), 1)
    oh = (ids == iota).astype(jnp.float32)
    hidden = jnp.dot(oh, e_ref[...], preferred_element_type=jnp.float32)
    hidden = hidden + pos_ref[...]
    out_ref[...] = jnp.dot(hidden, wt_ref[...],
                           preferred_element_type=jnp.float32)


def kernel(input_ids, embed_tokens, embed_positions, lm_head_w):
    ids_flat = input_ids.reshape(TOKENS, 1)
    pos_full = jnp.tile(embed_positions[:SEQ], (BB, 1))
    wt = lm_head_w.T

    grid = (TOKENS // TB,)
    out = pl.pallas_call(
        _body,
        grid=grid,
        in_specs=[
            pl.BlockSpec((TB, 1), lambda i: (i, 0)),
            pl.BlockSpec((VOCAB, HIDDEN), lambda i: (0, 0)),
            pl.BlockSpec((TB, HIDDEN), lambda i: (0, 0)),
            pl.BlockSpec((HIDDEN, VOCAB), lambda i: (0, 0)),
        ],
        out_specs=pl.BlockSpec((TB, VOCAB), lambda i: (i, 0)),
        out_shape=jax.ShapeDtypeStruct((TOKENS, VOCAB), jnp.float32),
    )(ids_flat, embed_tokens, pos_full, wt)
    return out.reshape(BATCH, SEQ, VOCAB)
